# baseline (speedup 1.0000x reference)
import functools

import jax
import jax.numpy as jnp
from jax import lax
from jax.experimental import pallas as pl
from jax.experimental.pallas import tpu as pltpu
from jax.experimental.pallas import tpu_sc as plsc

_VOCAB = 1000000
_DIM = 64
_BATCH = 16384
_CTX = 5

_NC = 2
_NS = 16
_NW = _NC * _NS
_BPW = _BATCH // _NW
_CHUNK = 128
_NCHUNK = _BPW // _CHUNK
_CROWS = _CHUNK * _CTX
_L = 16


def _skipgram_body(tgt_hbm, ctx_hbm, ttab_hbm, ctab_hbm, out_hbm,
                   tidx_v, trow_v, toff_v, cidx_v, crow_v, coff_v,
                   we_v, ce_v, out_v, sem):
    wid = lax.axis_index("s") * _NC + lax.axis_index("c")
    base = wid * _BPW

    lanes = lax.iota(jnp.int32, _L)
    lane0 = lanes == 0
    perms = [lanes ^ s for s in (8, 4, 2, 1)]

    def _shuf(x, perm):
        dims = lax.GatherDimensionNumbers(
            offset_dims=(), collapsed_slice_dims=(0,), start_index_map=(0,))
        return lax.gather(x, perm[:, None], dims, (1,),
                          mode=lax.GatherScatterMode.PROMISE_IN_BOUNDS)

    def chunk_body(ci, carry):
        b0 = base + ci * _CHUNK
        pltpu.sync_copy(tgt_hbm.at[pl.ds(b0, _CHUNK)], tidx_v)
        pltpu.sync_copy(ctx_hbm.at[pl.ds(b0 * _CTX, _CROWS)], cidx_v)

        def t_prep(i, c2):
            v = tidx_v[pl.ds(i * _L, _L)]
            trow_v[pl.ds(i * _L, _L)] = lax.shift_right_logical(v, 1)
            toff_v[pl.ds(i * _L, _L)] = (v & 1) * _DIM
            return c2

        def c_prep(i, c2):
            v = cidx_v[pl.ds(i * _L, _L)]
            crow_v[pl.ds(i * _L, _L)] = lax.shift_right_logical(v, 1)
            coff_v[pl.ds(i * _L, _L)] = (v & 1) * _DIM
            return c2

        lax.fori_loop(0, _CHUNK // _L, t_prep, 0)
        lax.fori_loop(0, _CROWS // _L, c_prep, 0)

        cps = [pltpu.async_copy(ttab_hbm.at[trow_v], we_v, sem)]
        for j in range(_CTX):
            cps.append(pltpu.async_copy(
                ctab_hbm.at[crow_v.at[pl.ds(j * _CHUNK, _CHUNK)]],
                ce_v.at[pl.ds(j * _CHUNK, _CHUNK)], sem))
        for cp in cps:
            cp.wait()

        def g_body(g, c2):
            tofs = toff_v[pl.ds(g * _L, _L)]
            cofs = [coff_v[pl.ds(g * (_L * _CTX) + _L * m, _L)]
                    for m in range(_CTX)]
            for k in range(_L):
                b = g * _L + k
                tof = tofs[k]
                we = [we_v[b, pl.ds(tof + _L * j, _L)] for j in range(4)]
                for c in range(_CTX):
                    q = k * _CTX + c
                    p = b * _CTX + c
                    cof = cofs[q // _L][q % _L]
                    acc = ce_v[p, pl.ds(cof, _L)] * we[0]
                    for j in range(1, 4):
                        acc = acc + ce_v[p, pl.ds(cof + _L * j, _L)] * we[j]
                    for perm in perms:
                        acc = acc + _shuf(acc, perm)
                    plsc.store_scatter(out_v,
                                       [jnp.full((_L,), p, jnp.int32)],
                                       acc, mask=lane0)
            return c2

        lax.fori_loop(0, _CHUNK // _L, g_body, 0)
        pltpu.sync_copy(out_v, out_hbm.at[pl.ds(b0 * _CTX, _CROWS)])
        return carry

    lax.fori_loop(0, _NCHUNK, chunk_body, 0)


@jax.jit
def _skipgram(tgt, ctx, ttab, ctab):
    mesh = plsc.VectorSubcoreMesh(core_axis_name="c", subcore_axis_name="s")
    f = functools.partial(
        pl.kernel,
        out_type=jax.ShapeDtypeStruct((_BATCH * _CTX,), jnp.float32),
        mesh=mesh,
        scratch_types=[
            pltpu.VMEM((_CHUNK,), jnp.int32),
            pltpu.VMEM((_CHUNK,), jnp.int32),
            pltpu.VMEM((_CHUNK,), jnp.int32),
            pltpu.VMEM((_CROWS,), jnp.int32),
            pltpu.VMEM((_CROWS,), jnp.int32),
            pltpu.VMEM((_CROWS,), jnp.int32),
            pltpu.VMEM((_CHUNK, 2 * _DIM), jnp.float32),
            pltpu.VMEM((_CROWS, 2 * _DIM), jnp.float32),
            pltpu.VMEM((_CROWS,), jnp.float32),
            pltpu.SemaphoreType.DMA,
        ],
        compiler_params=pltpu.CompilerParams(needs_layout_passes=False),
    )(_skipgram_body)
    return f(tgt, ctx, ttab, ctab).reshape(_BATCH, _CTX)


def kernel(target, context, target_table, context_table):
    tgt = jnp.asarray(target, jnp.int32).reshape(_BATCH)
    ctx = jnp.asarray(context, jnp.int32).reshape(_BATCH * _CTX)
    ttab = target_table.reshape(_VOCAB // 2, 2 * _DIM)
    ctab = context_table.reshape(_VOCAB // 2, 2 * _DIM)
    return _skipgram(tgt, ctx, ttab, ctab)
